# baseline (speedup 1.0000x reference)
import jax
import jax.numpy as jnp
from jax.experimental import pallas as pl
from jax.experimental.pallas import tpu as pltpu

_M, _K = 2, 3
_NNZ = 6


def _spmm_kernel(y_ref, xind_ref, xval_ref, out_ref):
    c = [[jnp.float32(0.0)] * _K for _ in range(_M)]
    for k in range(_NNZ):
        r = xind_ref[0, k]
        col = xind_ref[1, k]
        v = xval_ref[k]
        for i in range(_M):
            for j in range(_K):
                hit = jnp.logical_and(r == i, col == j)
                c[i][j] = c[i][j] + jnp.where(hit, v, jnp.float32(0.0))
    yb = y_ref[...]
    for i in range(_M):
        acc = c[i][0] * yb[0:1, :]
        for j in range(1, _K):
            acc = acc + c[i][j] * yb[j : j + 1, :]
        out_ref[i : i + 1, :] = acc


_NBLK = 2


def kernel(y, xind, xval):
    xind32 = xind.astype(jnp.int32)
    n = y.shape[1]
    blk = n // _NBLK
    return pl.pallas_call(
        _spmm_kernel,
        grid=(_NBLK,),
        out_shape=jax.ShapeDtypeStruct((_M, n), y.dtype),
        in_specs=[
            pl.BlockSpec((_K, blk), lambda i: (0, i),
                         memory_space=pltpu.VMEM),
            pl.BlockSpec(memory_space=pltpu.SMEM),
            pl.BlockSpec(memory_space=pltpu.SMEM),
        ],
        out_specs=pl.BlockSpec((_M, blk), lambda i: (0, i),
                               memory_space=pltpu.VMEM),
    )(y, xind32, xval)


# speedup vs baseline: 1.0893x; 1.0893x over previous
import jax
import jax.numpy as jnp
from jax.experimental import pallas as pl
from jax.experimental.pallas import tpu as pltpu

_M, _K = 2, 3
_NNZ = 6


def _spmm_kernel(y_ref, xind_ref, xval_ref, out_ref):
    c = [[jnp.float32(0.0)] * _K for _ in range(_M)]
    for k in range(_NNZ):
        r = xind_ref[0, k]
        col = xind_ref[1, k]
        v = xval_ref[k]
        for i in range(_M):
            for j in range(_K):
                hit = jnp.logical_and(r == i, col == j)
                c[i][j] = c[i][j] + jnp.where(hit, v, jnp.float32(0.0))
    yb = y_ref[...]
    for i in range(_M):
        acc = c[i][0] * yb[0:1, :]
        for j in range(1, _K):
            acc = acc + c[i][j] * yb[j : j + 1, :]
        out_ref[i : i + 1, :] = acc


def kernel(y, xind, xval):
    xind32 = xind.astype(jnp.int32)
    return pl.pallas_call(
        _spmm_kernel,
        out_shape=jax.ShapeDtypeStruct((_M, y.shape[1]), y.dtype),
        in_specs=[
            pl.BlockSpec(memory_space=pltpu.VMEM),
            pl.BlockSpec(memory_space=pltpu.SMEM),
            pl.BlockSpec(memory_space=pltpu.SMEM),
        ],
        out_specs=pl.BlockSpec(memory_space=pltpu.VMEM),
    )(y, xind32, xval)
